# Initial kernel scaffold; baseline (speedup 1.0000x reference)
import jax
import jax.numpy as jnp
from jax.experimental import pallas as pl


def kernel(x, gate_w, ew1, eb1, ew2, eb2, sw1, sb1, sw2, sb2):
    raise NotImplementedError("write your pallas kernel here")


# speedup vs baseline: 2.0370x; 2.0370x over previous
import functools

import jax
import jax.numpy as jnp
from jax import lax
from jax.experimental import pallas as pl
from jax.experimental.pallas import tpu as pltpu
from jax.experimental.pallas import tpu_sc as plsc

F32 = jnp.float32
I32 = jnp.int32

BLK = 256
GATE_BLK = 1024
NW = 32
DISP_CHUNK = 64
COMB_CHUNK = 32


def _gate_body(x_ref, gw_ref, wf_ref):
    x = x_ref[...]
    logits = lax.dot_general(x, gw_ref[...], (((1,), (1,)), ((), ())),
                             preferred_element_type=F32)
    gb, e = logits.shape
    m = jnp.max(logits, axis=-1, keepdims=True)
    ex = jnp.exp(logits - m)
    scores = ex / jnp.sum(ex, axis=-1, keepdims=True)
    ii = lax.broadcasted_iota(I32, (gb, e), 1)
    v1 = jnp.max(scores, axis=-1, keepdims=True)
    i1 = jnp.min(jnp.where(scores == v1, ii, e), axis=-1, keepdims=True)
    rest = jnp.where(ii == i1, -1.0, scores)
    v2 = jnp.max(rest, axis=-1, keepdims=True)
    i2 = jnp.min(jnp.where(rest == v2, ii, e), axis=-1, keepdims=True)
    wf = jnp.where(ii == i1, v1, 0.0) + jnp.where(ii == i2, v2, 0.0)
    wf_ref[...] = wf / (v1 + v2)


def _gating(x_flat, gate_w):
    n, hid = x_flat.shape
    e = gate_w.shape[0]
    return pl.pallas_call(
        _gate_body,
        grid=(n // GATE_BLK,),
        in_specs=[
            pl.BlockSpec((GATE_BLK, hid), lambda b: (b, 0)),
            pl.BlockSpec((e, hid), lambda b: (0, 0)),
        ],
        out_specs=pl.BlockSpec((GATE_BLK, e), lambda b: (b, 0)),
        out_shape=jax.ShapeDtypeStruct((n, e), F32),
    )(x_flat, gate_w)


def _moe_body(be_ref, x_ref, w1_ref, b1_ref, w2_ref, b2_ref, ws_ref, y_ref):
    del be_ref
    x = x_ref[...]
    h = jnp.dot(x, w1_ref[0], preferred_element_type=F32) + b1_ref[0]
    h = 0.5 * h * (1.0 + lax.erf(h * 0.7071067811865476))
    y = jnp.dot(h, w2_ref[0], preferred_element_type=F32) + b2_ref[0]
    y_ref[...] = y * ws_ref[...]


def _grouped_mlp(x_sorted, block_expert, w1_all, b1_all, w2_all, b2_all, w_ext):
    capx, hid = x_sorted.shape
    ne, _, inter = w1_all.shape
    nb = capx // BLK
    grid_spec = pltpu.PrefetchScalarGridSpec(
        num_scalar_prefetch=1,
        grid=(nb,),
        in_specs=[
            pl.BlockSpec((BLK, hid), lambda b, be: (b, 0)),
            pl.BlockSpec((1, hid, inter), lambda b, be: (be[b], 0, 0)),
            pl.BlockSpec((1, 1, inter), lambda b, be: (be[b], 0, 0)),
            pl.BlockSpec((1, inter, hid), lambda b, be: (be[b], 0, 0)),
            pl.BlockSpec((1, 1, hid), lambda b, be: (be[b], 0, 0)),
            pl.BlockSpec((BLK, 1), lambda b, be: (b, 0)),
        ],
        out_specs=pl.BlockSpec((BLK, hid), lambda b, be: (b, 0)),
    )
    return pl.pallas_call(
        _moe_body,
        grid_spec=grid_spec,
        out_shape=jax.ShapeDtypeStruct((capx, hid), F32),
    )(block_expert, x_sorted, w1_all, b1_all, w2_all, b2_all, w_ext)


def _dispatch(x_flat, tok_ext):
    n, hid = x_flat.shape
    capx = tok_ext.shape[0]
    per_w = capx // NW
    nch = per_w // DISP_CHUNK
    mesh = plsc.VectorSubcoreMesh(core_axis_name="c", subcore_axis_name="s")

    @functools.partial(
        pl.kernel,
        mesh=mesh,
        out_type=jax.ShapeDtypeStruct((capx, hid), F32),
        scratch_types=[
            pltpu.VMEM((DISP_CHUNK,), I32),
            pltpu.VMEM((DISP_CHUNK, hid), F32),
            pltpu.SemaphoreType.DMA,
        ],
    )
    def disp(x_hbm, idx_hbm, out_hbm, idx_v, rows_v, sem):
        wid = lax.axis_index("s") * 2 + lax.axis_index("c")
        base = wid * per_w

        def body(i, carry):
            off = pl.multiple_of(base + i * DISP_CHUNK, DISP_CHUNK)
            pltpu.sync_copy(idx_hbm.at[pl.ds(off, DISP_CHUNK)], idx_v)
            pltpu.async_copy(x_hbm.at[idx_v], rows_v, sem).wait()
            pltpu.sync_copy(rows_v, out_hbm.at[pl.ds(off, DISP_CHUNK)])
            return carry

        lax.fori_loop(0, nch, body, 0)

    return disp(x_flat, tok_ext)


def _combine(y_s, pos1, pos2, pos_sh):
    capx, hid = y_s.shape
    n = pos1.shape[0]
    per_w = n // NW
    nch = per_w // COMB_CHUNK
    mesh = plsc.VectorSubcoreMesh(core_axis_name="c", subcore_axis_name="s")
    nvec = hid // 16

    @functools.partial(
        pl.kernel,
        mesh=mesh,
        out_type=jax.ShapeDtypeStruct((n, hid), F32),
        scratch_types=[
            pltpu.VMEM((COMB_CHUNK,), I32),
            pltpu.VMEM((COMB_CHUNK,), I32),
            pltpu.VMEM((COMB_CHUNK,), I32),
            pltpu.VMEM((COMB_CHUNK, hid), F32),
            pltpu.VMEM((COMB_CHUNK, hid), F32),
            pltpu.VMEM((COMB_CHUNK, hid), F32),
            pltpu.SemaphoreType.DMA,
        ],
    )
    def comb(y_hbm, p1_hbm, p2_hbm, psh_hbm, out_hbm,
             i1_v, i2_v, i3_v, r1_v, r2_v, r3_v, sem):
        wid = lax.axis_index("s") * 2 + lax.axis_index("c")
        base = wid * per_w

        def body(i, carry):
            off = pl.multiple_of(base + i * COMB_CHUNK, COMB_CHUNK)
            pltpu.sync_copy(p1_hbm.at[pl.ds(off, COMB_CHUNK)], i1_v)
            pltpu.sync_copy(p2_hbm.at[pl.ds(off, COMB_CHUNK)], i2_v)
            pltpu.sync_copy(psh_hbm.at[pl.ds(off, COMB_CHUNK)], i3_v)
            pltpu.async_copy(y_hbm.at[i1_v], r1_v, sem).wait()
            pltpu.async_copy(y_hbm.at[i2_v], r2_v, sem).wait()
            pltpu.async_copy(y_hbm.at[i3_v], r3_v, sem).wait()

            def add_row(r, c):
                def add_vec(j, c2):
                    sl = pl.ds(j * 16, 16)
                    r1_v[r, sl] = r1_v[r, sl] + r2_v[r, sl] + r3_v[r, sl]
                    return c2
                return lax.fori_loop(0, nvec, add_vec, c)

            lax.fori_loop(0, COMB_CHUNK, add_row, 0)
            pltpu.sync_copy(r1_v, out_hbm.at[pl.ds(off, COMB_CHUNK)])
            return carry

        lax.fori_loop(0, nch, body, 0)

    return comb(y_s, pos1, pos2, pos_sh)


def kernel(x, gate_w, ew1, eb1, ew2, eb2, sw1, sb1, sw2, sb2):
    b, t, h, w, hid = x.shape
    n = b * t * h * w
    e = gate_w.shape[0]
    inter = ew1.shape[2]
    x_flat = x.reshape(n, hid)

    w_full = _gating(x_flat, gate_w)

    v1 = jnp.max(w_full, axis=-1)
    i1 = jnp.argmax(w_full, axis=-1).astype(I32)
    rest = jnp.where(jnp.arange(e)[None, :] == i1[:, None], 0.0, w_full)
    v2 = jnp.max(rest, axis=-1)
    i2 = jnp.argmax(rest, axis=-1).astype(I32)

    ids = jnp.concatenate([i1, i2])
    toks = jnp.tile(jnp.arange(n, dtype=I32), 2)
    wts = jnp.concatenate([v1, v2])
    order = jnp.argsort(ids, stable=True)
    ids_s = ids[order]
    counts = jnp.sum(ids[None, :] == jnp.arange(e)[:, None], axis=1)
    blocks_e = (counts + BLK - 1) // BLK
    block_base = BLK * (jnp.cumsum(blocks_e) - blocks_e)
    start_e = jnp.cumsum(counts) - counts
    rank = jnp.arange(2 * n, dtype=I32) - start_e[ids_s].astype(I32)
    pos_sorted = block_base[ids_s].astype(I32) + rank

    cap = 2 * n + e * BLK
    capx = cap + n
    tok_pad = jnp.zeros((cap,), I32).at[pos_sorted].set(toks[order])
    w_pad = jnp.zeros((cap,), F32).at[pos_sorted].set(wts[order])
    pos_pair = jnp.zeros((2 * n,), I32).at[order].set(pos_sorted)
    pos1, pos2 = pos_pair[:n], pos_pair[n:]
    pos_sh = cap + jnp.arange(n, dtype=I32)

    nbr = cap // BLK
    seg_ends = jnp.cumsum(blocks_e)
    block_expert = jnp.searchsorted(seg_ends, jnp.arange(nbr),
                                    side="right").astype(I32)
    block_expert = jnp.concatenate(
        [block_expert, jnp.full((n // BLK,), e, I32)])

    tok_ext = jnp.concatenate([tok_pad, jnp.arange(n, dtype=I32)])
    w_ext = jnp.concatenate([w_pad, jnp.ones((n,), F32)]).reshape(capx, 1)

    w1_all = jnp.concatenate([ew1, sw1[None]], axis=0)
    b1_all = jnp.concatenate([eb1, sb1[None]], axis=0)[:, None, :]
    w2_all = jnp.concatenate([ew2, sw2[None]], axis=0)
    b2_all = jnp.concatenate([eb2, sb2[None]], axis=0)[:, None, :]

    x_sorted = _dispatch(x_flat, tok_ext)

    y_s = _grouped_mlp(x_sorted, block_expert, w1_all, b1_all, w2_all,
                       b2_all, w_ext)

    out = _combine(y_s, pos1, pos2, pos_sh)
    return out.reshape(b, t, h, w, hid)


# speedup vs baseline: 2.1042x; 1.0330x over previous
import functools

import jax
import jax.numpy as jnp
from jax import lax
from jax.experimental import pallas as pl
from jax.experimental.pallas import tpu as pltpu
from jax.experimental.pallas import tpu_sc as plsc

F32 = jnp.float32
I32 = jnp.int32

BLK = 256
GATE_BLK = 1024
NW = 32
DISP_CHUNK = 64
COMB_CHUNK = 32


def _gate_body(x_ref, gw_ref, wf_ref):
    x = x_ref[...]
    logits = lax.dot_general(x, gw_ref[...], (((1,), (1,)), ((), ())),
                             preferred_element_type=F32)
    gb, e = logits.shape
    m = jnp.max(logits, axis=-1, keepdims=True)
    ex = jnp.exp(logits - m)
    scores = ex / jnp.sum(ex, axis=-1, keepdims=True)
    ii = lax.broadcasted_iota(I32, (gb, e), 1)
    v1 = jnp.max(scores, axis=-1, keepdims=True)
    i1 = jnp.min(jnp.where(scores == v1, ii, e), axis=-1, keepdims=True)
    rest = jnp.where(ii == i1, -1.0, scores)
    v2 = jnp.max(rest, axis=-1, keepdims=True)
    i2 = jnp.min(jnp.where(rest == v2, ii, e), axis=-1, keepdims=True)
    wf = jnp.where(ii == i1, v1, 0.0) + jnp.where(ii == i2, v2, 0.0)
    wf_ref[...] = wf / (v1 + v2)


def _gating(x_flat, gate_w):
    n, hid = x_flat.shape
    e = gate_w.shape[0]
    return pl.pallas_call(
        _gate_body,
        grid=(n // GATE_BLK,),
        in_specs=[
            pl.BlockSpec((GATE_BLK, hid), lambda b: (b, 0)),
            pl.BlockSpec((e, hid), lambda b: (0, 0)),
        ],
        out_specs=pl.BlockSpec((GATE_BLK, e), lambda b: (b, 0)),
        out_shape=jax.ShapeDtypeStruct((n, e), F32),
    )(x_flat, gate_w)


def _moe_body(be_ref, x_ref, w1_ref, b1_ref, w2_ref, b2_ref, ws_ref, y_ref):
    del be_ref
    x = x_ref[...].astype(jnp.bfloat16)
    h = jnp.dot(x, w1_ref[0], preferred_element_type=F32) + b1_ref[0]
    h = 0.5 * h * (1.0 + lax.erf(h * 0.7071067811865476))
    y = jnp.dot(h.astype(jnp.bfloat16), w2_ref[0],
                preferred_element_type=F32) + b2_ref[0]
    y_ref[...] = y * ws_ref[...]


def _grouped_mlp(x_sorted, block_expert, w1_all, b1_all, w2_all, b2_all, w_ext):
    capx, hid = x_sorted.shape
    ne, _, inter = w1_all.shape
    nb = capx // BLK
    grid_spec = pltpu.PrefetchScalarGridSpec(
        num_scalar_prefetch=1,
        grid=(nb,),
        in_specs=[
            pl.BlockSpec((BLK, hid), lambda b, be: (b, 0)),
            pl.BlockSpec((1, hid, inter), lambda b, be: (be[b], 0, 0)),
            pl.BlockSpec((1, 1, inter), lambda b, be: (be[b], 0, 0)),
            pl.BlockSpec((1, inter, hid), lambda b, be: (be[b], 0, 0)),
            pl.BlockSpec((1, 1, hid), lambda b, be: (be[b], 0, 0)),
            pl.BlockSpec((BLK, 1), lambda b, be: (b, 0)),
        ],
        out_specs=pl.BlockSpec((BLK, hid), lambda b, be: (b, 0)),
    )
    return pl.pallas_call(
        _moe_body,
        grid_spec=grid_spec,
        out_shape=jax.ShapeDtypeStruct((capx, hid), F32),
    )(block_expert, x_sorted, w1_all, b1_all, w2_all, b2_all, w_ext)


def _shared_body(x_ref, w1_ref, b1_ref, w2_ref, b2_ref, y_ref):
    x = x_ref[...].astype(jnp.bfloat16)
    h = jnp.dot(x, w1_ref[...], preferred_element_type=F32) + b1_ref[...]
    h = 0.5 * h * (1.0 + lax.erf(h * 0.7071067811865476))
    y_ref[...] = jnp.dot(h.astype(jnp.bfloat16), w2_ref[...],
                         preferred_element_type=F32) + b2_ref[...]


def _shared_mlp(x_flat, sw1, sb1, sw2, sb2):
    n, hid = x_flat.shape
    inter = sw1.shape[1]
    return pl.pallas_call(
        _shared_body,
        grid=(n // BLK,),
        in_specs=[
            pl.BlockSpec((BLK, hid), lambda b: (b, 0)),
            pl.BlockSpec((hid, inter), lambda b: (0, 0)),
            pl.BlockSpec((1, inter), lambda b: (0, 0)),
            pl.BlockSpec((inter, hid), lambda b: (0, 0)),
            pl.BlockSpec((1, hid), lambda b: (0, 0)),
        ],
        out_specs=pl.BlockSpec((BLK, hid), lambda b: (b, 0)),
        out_shape=jax.ShapeDtypeStruct((n, hid), F32),
    )(x_flat, sw1, sb1[None, :], sw2, sb2[None, :])


def _dispatch(x_flat, tok_ext):
    n, hid = x_flat.shape
    capx = tok_ext.shape[0]
    per_w = capx // NW
    nch = per_w // DISP_CHUNK
    mesh = plsc.VectorSubcoreMesh(core_axis_name="c", subcore_axis_name="s")

    @functools.partial(
        pl.kernel,
        mesh=mesh,
        out_type=jax.ShapeDtypeStruct((capx, hid), F32),
        scratch_types=[
            pltpu.VMEM((DISP_CHUNK,), I32),
            pltpu.VMEM((DISP_CHUNK, hid), F32),
            pltpu.SemaphoreType.DMA,
        ],
    )
    def disp(x_hbm, idx_hbm, out_hbm, idx_v, rows_v, sem):
        wid = lax.axis_index("s") * 2 + lax.axis_index("c")
        base = wid * per_w

        def body(i, carry):
            off = pl.multiple_of(base + i * DISP_CHUNK, DISP_CHUNK)
            pltpu.sync_copy(idx_hbm.at[pl.ds(off, DISP_CHUNK)], idx_v)
            pltpu.async_copy(x_hbm.at[idx_v], rows_v, sem).wait()
            pltpu.sync_copy(rows_v, out_hbm.at[pl.ds(off, DISP_CHUNK)])
            return carry

        lax.fori_loop(0, nch, body, 0)

    return disp(x_flat, tok_ext)


def _combine(y_s, y_shared, pos1, pos2):
    capx, hid = y_s.shape
    n = pos1.shape[0]
    per_w = n // NW
    nch = per_w // COMB_CHUNK
    mesh = plsc.VectorSubcoreMesh(core_axis_name="c", subcore_axis_name="s")
    nvec = hid // 16

    @functools.partial(
        pl.kernel,
        mesh=mesh,
        out_type=jax.ShapeDtypeStruct((n, hid), F32),
        scratch_types=[
            pltpu.VMEM((COMB_CHUNK,), I32),
            pltpu.VMEM((COMB_CHUNK,), I32),
            pltpu.VMEM((COMB_CHUNK, hid), F32),
            pltpu.VMEM((COMB_CHUNK, hid), F32),
            pltpu.VMEM((COMB_CHUNK, hid), F32),
            pltpu.SemaphoreType.DMA,
        ],
    )
    def comb(y_hbm, ysh_hbm, p1_hbm, p2_hbm, out_hbm,
             i1_v, i2_v, r1_v, r2_v, r3_v, sem):
        wid = lax.axis_index("s") * 2 + lax.axis_index("c")
        base = wid * per_w

        def body(i, carry):
            off = pl.multiple_of(base + i * COMB_CHUNK, COMB_CHUNK)
            pltpu.sync_copy(p1_hbm.at[pl.ds(off, COMB_CHUNK)], i1_v)
            pltpu.sync_copy(p2_hbm.at[pl.ds(off, COMB_CHUNK)], i2_v)
            pltpu.sync_copy(ysh_hbm.at[pl.ds(off, COMB_CHUNK)], r3_v)
            pltpu.async_copy(y_hbm.at[i1_v], r1_v, sem).wait()
            pltpu.async_copy(y_hbm.at[i2_v], r2_v, sem).wait()

            def add_row(r, c):
                def add_vec(j, c2):
                    sl = pl.ds(j * 16, 16)
                    r1_v[r, sl] = r1_v[r, sl] + r2_v[r, sl] + r3_v[r, sl]
                    return c2
                return lax.fori_loop(0, nvec, add_vec, c)

            lax.fori_loop(0, COMB_CHUNK, add_row, 0)
            pltpu.sync_copy(r1_v, out_hbm.at[pl.ds(off, COMB_CHUNK)])
            return carry

        lax.fori_loop(0, nch, body, 0)

    return comb(y_s, y_shared, pos1, pos2)


def kernel(x, gate_w, ew1, eb1, ew2, eb2, sw1, sb1, sw2, sb2):
    b, t, h, w, hid = x.shape
    n = b * t * h * w
    e = gate_w.shape[0]
    inter = ew1.shape[2]
    x_flat = x.reshape(n, hid)

    w_full = _gating(x_flat, gate_w)

    v1 = jnp.max(w_full, axis=-1)
    i1 = jnp.argmax(w_full, axis=-1).astype(I32)
    rest = jnp.where(jnp.arange(e)[None, :] == i1[:, None], 0.0, w_full)
    v2 = jnp.max(rest, axis=-1)
    i2 = jnp.argmax(rest, axis=-1).astype(I32)

    ids = jnp.concatenate([i1, i2])
    toks = jnp.tile(jnp.arange(n, dtype=I32), 2)
    wts = jnp.concatenate([v1, v2])
    order = jnp.argsort(ids, stable=True)
    ids_s = ids[order]
    counts = jnp.sum(ids[None, :] == jnp.arange(e)[:, None], axis=1)
    blocks_e = (counts + BLK - 1) // BLK
    block_base = BLK * (jnp.cumsum(blocks_e) - blocks_e)
    start_e = jnp.cumsum(counts) - counts
    rank = jnp.arange(2 * n, dtype=I32) - start_e[ids_s].astype(I32)
    pos_sorted = block_base[ids_s].astype(I32) + rank

    cap = 2 * n + e * BLK
    tok_pad = jnp.zeros((cap,), I32).at[pos_sorted].set(toks[order])
    w_pad = jnp.zeros((cap,), F32).at[pos_sorted].set(wts[order])
    pos_pair = jnp.zeros((2 * n,), I32).at[order].set(pos_sorted)
    pos1, pos2 = pos_pair[:n], pos_pair[n:]

    nbr = cap // BLK
    seg_ends = jnp.cumsum(blocks_e)
    block_expert = jnp.minimum(
        jnp.searchsorted(seg_ends, jnp.arange(nbr), side="right"),
        e - 1).astype(I32)
    w_ext = w_pad.reshape(cap, 1)

    bf16 = jnp.bfloat16
    w1_all = ew1.astype(bf16)
    b1_all = eb1[:, None, :]
    w2_all = ew2.astype(bf16)
    b2_all = eb2[:, None, :]

    x_sorted = _dispatch(x_flat, tok_pad)

    y_s = _grouped_mlp(x_sorted, block_expert, w1_all, b1_all, w2_all,
                       b2_all, w_ext)
    y_sh = _shared_mlp(x_flat, sw1.astype(bf16), sb1, sw2.astype(bf16),
                       sb2)

    out = _combine(y_s, y_sh, pos1, pos2)
    return out.reshape(b, t, h, w, hid)


# speedup vs baseline: 2.7551x; 1.3093x over previous
import functools

import jax
import jax.numpy as jnp
from jax import lax
from jax.experimental import pallas as pl
from jax.experimental.pallas import tpu as pltpu
from jax.experimental.pallas import tpu_sc as plsc

F32 = jnp.float32
I32 = jnp.int32

BLK = 256
GATE_BLK = 1024
NW = 32
DISP_CHUNK = 64
COMB_CHUNK = 32


def _gate_body(x_ref, gw_ref, wf_ref):
    x = x_ref[...]
    logits = lax.dot_general(x, gw_ref[...], (((1,), (1,)), ((), ())),
                             preferred_element_type=F32)
    gb, e = logits.shape
    m = jnp.max(logits, axis=-1, keepdims=True)
    ex = jnp.exp(logits - m)
    scores = ex / jnp.sum(ex, axis=-1, keepdims=True)
    ii = lax.broadcasted_iota(I32, (gb, e), 1)
    v1 = jnp.max(scores, axis=-1, keepdims=True)
    i1 = jnp.min(jnp.where(scores == v1, ii, e), axis=-1, keepdims=True)
    rest = jnp.where(ii == i1, -1.0, scores)
    v2 = jnp.max(rest, axis=-1, keepdims=True)
    i2 = jnp.min(jnp.where(rest == v2, ii, e), axis=-1, keepdims=True)
    wf = jnp.where(ii == i1, v1, 0.0) + jnp.where(ii == i2, v2, 0.0)
    wf_ref[...] = wf / (v1 + v2)


def _gating(x_flat, gate_w):
    n, hid = x_flat.shape
    e = gate_w.shape[0]
    return pl.pallas_call(
        _gate_body,
        grid=(n // GATE_BLK,),
        in_specs=[
            pl.BlockSpec((GATE_BLK, hid), lambda b: (b, 0)),
            pl.BlockSpec((e, hid), lambda b: (0, 0)),
        ],
        out_specs=pl.BlockSpec((GATE_BLK, e), lambda b: (b, 0)),
        out_shape=jax.ShapeDtypeStruct((n, e), F32),
    )(x_flat, gate_w)


def _moe_body(be_ref, x_ref, w1_ref, b1_ref, w2_ref, b2_ref, ws_ref, y_ref):
    del be_ref
    x = x_ref[...].astype(jnp.bfloat16)
    h = jnp.dot(x, w1_ref[0], preferred_element_type=F32) + b1_ref[0]
    h = 0.5 * h * (1.0 + lax.erf(h * 0.7071067811865476))
    y = jnp.dot(h.astype(jnp.bfloat16), w2_ref[0],
                preferred_element_type=F32) + b2_ref[0]
    y_ref[...] = y * ws_ref[...]


def _grouped_mlp(x_sorted, block_expert, w1_all, b1_all, w2_all, b2_all, w_ext):
    capx, hid = x_sorted.shape
    ne, _, inter = w1_all.shape
    nb = capx // BLK
    grid_spec = pltpu.PrefetchScalarGridSpec(
        num_scalar_prefetch=1,
        grid=(nb,),
        in_specs=[
            pl.BlockSpec((BLK, hid), lambda b, be: (b, 0)),
            pl.BlockSpec((1, hid, inter), lambda b, be: (be[b], 0, 0)),
            pl.BlockSpec((1, 1, inter), lambda b, be: (be[b], 0, 0)),
            pl.BlockSpec((1, inter, hid), lambda b, be: (be[b], 0, 0)),
            pl.BlockSpec((1, 1, hid), lambda b, be: (be[b], 0, 0)),
            pl.BlockSpec((BLK, 1), lambda b, be: (b, 0)),
        ],
        out_specs=pl.BlockSpec((BLK, hid), lambda b, be: (b, 0)),
    )
    return pl.pallas_call(
        _moe_body,
        grid_spec=grid_spec,
        out_shape=jax.ShapeDtypeStruct((capx, hid), F32),
    )(block_expert, x_sorted, w1_all, b1_all, w2_all, b2_all, w_ext)


def _shared_body(x_ref, w1_ref, b1_ref, w2_ref, b2_ref, y_ref):
    x = x_ref[...].astype(jnp.bfloat16)
    h = jnp.dot(x, w1_ref[...], preferred_element_type=F32) + b1_ref[...]
    h = 0.5 * h * (1.0 + lax.erf(h * 0.7071067811865476))
    y_ref[...] = jnp.dot(h.astype(jnp.bfloat16), w2_ref[...],
                         preferred_element_type=F32) + b2_ref[...]


def _shared_mlp(x_flat, sw1, sb1, sw2, sb2):
    n, hid = x_flat.shape
    inter = sw1.shape[1]
    return pl.pallas_call(
        _shared_body,
        grid=(n // BLK,),
        in_specs=[
            pl.BlockSpec((BLK, hid), lambda b: (b, 0)),
            pl.BlockSpec((hid, inter), lambda b: (0, 0)),
            pl.BlockSpec((1, inter), lambda b: (0, 0)),
            pl.BlockSpec((inter, hid), lambda b: (0, 0)),
            pl.BlockSpec((1, hid), lambda b: (0, 0)),
        ],
        out_specs=pl.BlockSpec((BLK, hid), lambda b: (b, 0)),
        out_shape=jax.ShapeDtypeStruct((n, hid), F32),
    )(x_flat, sw1, sb1[None, :], sw2, sb2[None, :])


def _dispatch(x_flat, pos_pair, cap):
    n, hid = x_flat.shape
    m2 = pos_pair.shape[0]
    per_w = m2 // NW
    nch = per_w // DISP_CHUNK
    mesh = plsc.VectorSubcoreMesh(core_axis_name="c", subcore_axis_name="s")

    @functools.partial(
        pl.kernel,
        mesh=mesh,
        out_type=jax.ShapeDtypeStruct((cap, hid), F32),
        scratch_types=[
            pltpu.VMEM((DISP_CHUNK,), I32),
            pltpu.VMEM((DISP_CHUNK, hid), F32),
            pltpu.SemaphoreType.DMA,
        ],
    )
    def disp(x_hbm, pos_hbm, out_hbm, idx_v, rows_v, sem):
        wid = lax.axis_index("s") * 2 + lax.axis_index("c")
        base = wid * per_w

        def body(i, carry):
            off = pl.multiple_of(base + i * DISP_CHUNK, DISP_CHUNK)
            xoff = pl.multiple_of(lax.rem(off, n), DISP_CHUNK)
            pltpu.sync_copy(pos_hbm.at[pl.ds(off, DISP_CHUNK)], idx_v)
            pltpu.sync_copy(x_hbm.at[pl.ds(xoff, DISP_CHUNK)], rows_v)
            pltpu.async_copy(rows_v, out_hbm.at[idx_v], sem).wait()
            return carry

        lax.fori_loop(0, nch, body, 0)

    return disp(x_flat, pos_pair)


def _combine(y_s, y_shared, pos1, pos2):
    capx, hid = y_s.shape
    n = pos1.shape[0]
    per_w = n // NW
    nch = per_w // COMB_CHUNK
    mesh = plsc.VectorSubcoreMesh(core_axis_name="c", subcore_axis_name="s")
    nvec = hid // 16

    @functools.partial(
        pl.kernel,
        mesh=mesh,
        out_type=jax.ShapeDtypeStruct((n, hid), F32),
        scratch_types=[
            pltpu.VMEM((COMB_CHUNK,), I32),
            pltpu.VMEM((COMB_CHUNK,), I32),
            pltpu.VMEM((COMB_CHUNK, hid), F32),
            pltpu.VMEM((COMB_CHUNK, hid), F32),
            pltpu.VMEM((COMB_CHUNK, hid), F32),
            pltpu.SemaphoreType.DMA,
        ],
    )
    def comb(y_hbm, ysh_hbm, p1_hbm, p2_hbm, out_hbm,
             i1_v, i2_v, r1_v, r2_v, r3_v, sem):
        wid = lax.axis_index("s") * 2 + lax.axis_index("c")
        base = wid * per_w

        def body(i, carry):
            off = pl.multiple_of(base + i * COMB_CHUNK, COMB_CHUNK)
            pltpu.sync_copy(p1_hbm.at[pl.ds(off, COMB_CHUNK)], i1_v)
            pltpu.sync_copy(p2_hbm.at[pl.ds(off, COMB_CHUNK)], i2_v)
            pltpu.sync_copy(ysh_hbm.at[pl.ds(off, COMB_CHUNK)], r3_v)
            pltpu.async_copy(y_hbm.at[i1_v], r1_v, sem).wait()
            pltpu.async_copy(y_hbm.at[i2_v], r2_v, sem).wait()

            def add_row(r, c):
                def add_vec(j, c2):
                    sl = pl.ds(j * 16, 16)
                    r1_v[r, sl] = r1_v[r, sl] + r2_v[r, sl] + r3_v[r, sl]
                    return c2
                return lax.fori_loop(0, nvec, add_vec, c)

            lax.fori_loop(0, COMB_CHUNK, add_row, 0)
            pltpu.sync_copy(r1_v, out_hbm.at[pl.ds(off, COMB_CHUNK)])
            return carry

        lax.fori_loop(0, nch, body, 0)

    return comb(y_s, y_shared, pos1, pos2)


def kernel(x, gate_w, ew1, eb1, ew2, eb2, sw1, sb1, sw2, sb2):
    b, t, h, w, hid = x.shape
    n = b * t * h * w
    e = gate_w.shape[0]
    inter = ew1.shape[2]
    x_flat = x.reshape(n, hid)

    w_full = _gating(x_flat, gate_w)

    v1 = jnp.max(w_full, axis=-1)
    i1 = jnp.argmax(w_full, axis=-1).astype(I32)
    rest = jnp.where(jnp.arange(e)[None, :] == i1[:, None], 0.0, w_full)
    v2 = jnp.max(rest, axis=-1)
    i2 = jnp.argmax(rest, axis=-1).astype(I32)

    ids = jnp.concatenate([i1, i2])
    wts = jnp.concatenate([v1, v2])
    onehot = (ids[:, None] == jnp.arange(e)[None, :]).astype(I32)
    rank_all = jnp.cumsum(onehot, axis=0) - onehot
    rank = jnp.take_along_axis(rank_all, ids[:, None], axis=1)[:, 0]
    counts = rank_all[-1] + onehot[-1]
    blocks_e = (counts + BLK - 1) // BLK
    block_base = BLK * (jnp.cumsum(blocks_e) - blocks_e)
    pos_pair = block_base[ids].astype(I32) + rank.astype(I32)
    pos1, pos2 = pos_pair[:n], pos_pair[n:]

    cap = 2 * n + e * BLK
    w_pad = jnp.zeros((cap,), F32).at[pos_pair].set(
        wts, unique_indices=True, mode="drop")

    nbr = cap // BLK
    seg_ends = jnp.cumsum(blocks_e)
    block_expert = jnp.minimum(
        jnp.searchsorted(seg_ends, jnp.arange(nbr), side="right"),
        e - 1).astype(I32)
    w_ext = w_pad.reshape(cap, 1)

    bf16 = jnp.bfloat16
    w1_all = ew1.astype(bf16)
    b1_all = eb1[:, None, :]
    w2_all = ew2.astype(bf16)
    b2_all = eb2[:, None, :]

    x_sorted = _dispatch(x_flat, pos_pair, cap)

    y_s = _grouped_mlp(x_sorted, block_expert, w1_all, b1_all, w2_all,
                       b2_all, w_ext)
    y_sh = _shared_mlp(x_flat, sw1.astype(bf16), sb1, sw2.astype(bf16),
                       sb2)

    out = _combine(y_s, y_sh, pos1, pos2)
    return out.reshape(b, t, h, w, hid)


# speedup vs baseline: 2.9214x; 1.0604x over previous
import functools

import jax
import jax.numpy as jnp
from jax import lax
from jax.experimental import pallas as pl
from jax.experimental.pallas import tpu as pltpu
from jax.experimental.pallas import tpu_sc as plsc

F32 = jnp.float32
I32 = jnp.int32

BLK = 256
GATE_BLK = 1024
NW = 32
DISP_CHUNK = 64
COMB_CHUNK = 32


def _gate_body(x_ref, gw_ref, i1_ref, v1_ref, r1_ref, i2_ref, v2_ref, r2_ref,
               t1_ref, t2_ref, c1_ref, c2_ref):
    b = pl.program_id(0)

    @pl.when(b == 0)
    def _init():
        c1_ref[...] = jnp.zeros_like(c1_ref)
        c2_ref[...] = jnp.zeros_like(c2_ref)

    x = x_ref[...]
    logits = lax.dot_general(x, gw_ref[...], (((1,), (1,)), ((), ())),
                             preferred_element_type=F32)
    gb, e = logits.shape
    m = jnp.max(logits, axis=-1, keepdims=True)
    ex = jnp.exp(logits - m)
    scores = ex / jnp.sum(ex, axis=-1, keepdims=True)
    ii = lax.broadcasted_iota(I32, (gb, e), 1)
    v1 = jnp.max(scores, axis=-1, keepdims=True)
    i1 = jnp.min(jnp.where(scores == v1, ii, e), axis=-1, keepdims=True)
    rest = jnp.where(ii == i1, -1.0, scores)
    v2 = jnp.max(rest, axis=-1, keepdims=True)
    i2 = jnp.min(jnp.where(rest == v2, ii, e), axis=-1, keepdims=True)
    wsum = v1 + v2

    onehot1 = (ii == i1).astype(F32)
    onehot2 = (ii == i2).astype(F32)
    lt = (lax.broadcasted_iota(I32, (gb, gb), 0)
          > lax.broadcasted_iota(I32, (gb, gb), 1)).astype(F32)
    prefix1 = jnp.dot(lt, onehot1, preferred_element_type=F32)
    prefix2 = jnp.dot(lt, onehot2, preferred_element_type=F32)
    c1 = c1_ref[...]
    c2 = c2_ref[...]
    rank1 = jnp.sum(jnp.where(onehot1 > 0, prefix1 + c1, 0.0),
                    axis=-1, keepdims=True)
    rank2 = jnp.sum(jnp.where(onehot2 > 0, prefix2 + c2, 0.0),
                    axis=-1, keepdims=True)
    c1_ref[...] = c1 + jnp.sum(onehot1, axis=0, keepdims=True)
    c2_ref[...] = c2 + jnp.sum(onehot2, axis=0, keepdims=True)

    i1_ref[...] = i1
    i2_ref[...] = i2
    v1_ref[...] = v1 / wsum
    v2_ref[...] = v2 / wsum
    r1_ref[...] = rank1.astype(I32)
    r2_ref[...] = rank2.astype(I32)
    t1_ref[...] = c1_ref[...]
    t2_ref[...] = c2_ref[...]


def _gating(x_flat, gate_w):
    n, hid = x_flat.shape
    e = gate_w.shape[0]
    col = lambda b: (b, 0)
    whole = lambda b: (0, 0)
    return pl.pallas_call(
        _gate_body,
        grid=(n // GATE_BLK,),
        in_specs=[
            pl.BlockSpec((GATE_BLK, hid), col),
            pl.BlockSpec((e, hid), whole),
        ],
        out_specs=[
            pl.BlockSpec((GATE_BLK, 1), col),
            pl.BlockSpec((GATE_BLK, 1), col),
            pl.BlockSpec((GATE_BLK, 1), col),
            pl.BlockSpec((GATE_BLK, 1), col),
            pl.BlockSpec((GATE_BLK, 1), col),
            pl.BlockSpec((GATE_BLK, 1), col),
            pl.BlockSpec((1, e), whole),
            pl.BlockSpec((1, e), whole),
        ],
        out_shape=[
            jax.ShapeDtypeStruct((n, 1), I32),
            jax.ShapeDtypeStruct((n, 1), F32),
            jax.ShapeDtypeStruct((n, 1), I32),
            jax.ShapeDtypeStruct((n, 1), I32),
            jax.ShapeDtypeStruct((n, 1), F32),
            jax.ShapeDtypeStruct((n, 1), I32),
            jax.ShapeDtypeStruct((1, e), F32),
            jax.ShapeDtypeStruct((1, e), F32),
        ],
        scratch_shapes=[
            pltpu.VMEM((1, e), F32),
            pltpu.VMEM((1, e), F32),
        ],
    )(x_flat, gate_w)


def _moe_body(be_ref, x_ref, w1_ref, b1_ref, w2_ref, b2_ref, ws_ref, y_ref):
    del be_ref
    x = x_ref[...].astype(jnp.bfloat16)
    h = jnp.dot(x, w1_ref[0], preferred_element_type=F32) + b1_ref[0]
    h = 0.5 * h * (1.0 + lax.erf(h * 0.7071067811865476))
    y = jnp.dot(h.astype(jnp.bfloat16), w2_ref[0],
                preferred_element_type=F32) + b2_ref[0]
    y_ref[...] = y * ws_ref[...]


def _grouped_mlp(x_sorted, block_expert, w1_all, b1_all, w2_all, b2_all, w_ext):
    capx, hid = x_sorted.shape
    ne, _, inter = w1_all.shape
    nb = capx // BLK
    grid_spec = pltpu.PrefetchScalarGridSpec(
        num_scalar_prefetch=1,
        grid=(nb,),
        in_specs=[
            pl.BlockSpec((BLK, hid), lambda b, be: (b, 0)),
            pl.BlockSpec((1, hid, inter), lambda b, be: (be[b], 0, 0)),
            pl.BlockSpec((1, 1, inter), lambda b, be: (be[b], 0, 0)),
            pl.BlockSpec((1, inter, hid), lambda b, be: (be[b], 0, 0)),
            pl.BlockSpec((1, 1, hid), lambda b, be: (be[b], 0, 0)),
            pl.BlockSpec((BLK, 1), lambda b, be: (b, 0)),
        ],
        out_specs=pl.BlockSpec((BLK, hid), lambda b, be: (b, 0)),
    )
    return pl.pallas_call(
        _moe_body,
        grid_spec=grid_spec,
        out_shape=jax.ShapeDtypeStruct((capx, hid), F32),
    )(block_expert, x_sorted, w1_all, b1_all, w2_all, b2_all, w_ext)


def _shared_body(x_ref, w1_ref, b1_ref, w2_ref, b2_ref, y_ref):
    x = x_ref[...].astype(jnp.bfloat16)
    h = jnp.dot(x, w1_ref[...], preferred_element_type=F32) + b1_ref[...]
    h = 0.5 * h * (1.0 + lax.erf(h * 0.7071067811865476))
    y_ref[...] = jnp.dot(h.astype(jnp.bfloat16), w2_ref[...],
                         preferred_element_type=F32) + b2_ref[...]


def _shared_mlp(x_flat, sw1, sb1, sw2, sb2):
    n, hid = x_flat.shape
    inter = sw1.shape[1]
    return pl.pallas_call(
        _shared_body,
        grid=(n // BLK,),
        in_specs=[
            pl.BlockSpec((BLK, hid), lambda b: (b, 0)),
            pl.BlockSpec((hid, inter), lambda b: (0, 0)),
            pl.BlockSpec((1, inter), lambda b: (0, 0)),
            pl.BlockSpec((inter, hid), lambda b: (0, 0)),
            pl.BlockSpec((1, hid), lambda b: (0, 0)),
        ],
        out_specs=pl.BlockSpec((BLK, hid), lambda b: (b, 0)),
        out_shape=jax.ShapeDtypeStruct((n, hid), F32),
    )(x_flat, sw1, sb1[None, :], sw2, sb2[None, :])


def _dispatch(x_flat, pos_pair, cap):
    n, hid = x_flat.shape
    m2 = pos_pair.shape[0]
    per_w = m2 // NW
    nch = per_w // DISP_CHUNK
    mesh = plsc.VectorSubcoreMesh(core_axis_name="c", subcore_axis_name="s")

    @functools.partial(
        pl.kernel,
        mesh=mesh,
        out_type=jax.ShapeDtypeStruct((cap, hid), F32),
        scratch_types=[
            pltpu.VMEM((DISP_CHUNK,), I32),
            pltpu.VMEM((DISP_CHUNK, hid), F32),
            pltpu.SemaphoreType.DMA,
        ],
    )
    def disp(x_hbm, pos_hbm, out_hbm, idx_v, rows_v, semx):
        wid = lax.axis_index("s") * 2 + lax.axis_index("c")
        base = wid * per_w

        def body(i, carry):
            off = pl.multiple_of(base + i * DISP_CHUNK, DISP_CHUNK)
            xoff = pl.multiple_of(lax.rem(off, n), DISP_CHUNK)
            pltpu.sync_copy(pos_hbm.at[pl.ds(off, DISP_CHUNK)], idx_v)
            pltpu.sync_copy(x_hbm.at[pl.ds(xoff, DISP_CHUNK)], rows_v)
            pltpu.async_copy(rows_v, out_hbm.at[idx_v], semx).wait()
            return carry

        lax.fori_loop(0, nch, body, 0)

    return disp(x_flat, pos_pair)


def _combine(y_s, y_shared, pos1, pos2):
    capx, hid = y_s.shape
    n = pos1.shape[0]
    per_w = n // NW
    nch = per_w // COMB_CHUNK
    mesh = plsc.VectorSubcoreMesh(core_axis_name="c", subcore_axis_name="s")
    nvec = hid // 16

    @functools.partial(
        pl.kernel,
        mesh=mesh,
        out_type=jax.ShapeDtypeStruct((n, hid), F32),
        scratch_types=[
            pltpu.VMEM((COMB_CHUNK,), I32),
            pltpu.VMEM((COMB_CHUNK,), I32),
            pltpu.VMEM((COMB_CHUNK, hid), F32),
            pltpu.VMEM((COMB_CHUNK, hid), F32),
            pltpu.VMEM((COMB_CHUNK, hid), F32),
            pltpu.SemaphoreType.DMA,
            pltpu.SemaphoreType.DMA,
            pltpu.SemaphoreType.DMA,
        ],
    )
    def comb(y_hbm, ysh_hbm, p1_hbm, p2_hbm, out_hbm,
             i1_v, i2_v, r1_v, r2_v, r3_v, s1, s2, s3):
        wid = lax.axis_index("s") * 2 + lax.axis_index("c")
        base = wid * per_w

        def body(i, carry):
            off = pl.multiple_of(base + i * COMB_CHUNK, COMB_CHUNK)
            pltpu.sync_copy(p1_hbm.at[pl.ds(off, COMB_CHUNK)], i1_v)
            pltpu.sync_copy(p2_hbm.at[pl.ds(off, COMB_CHUNK)], i2_v)
            d3 = pltpu.async_copy(ysh_hbm.at[pl.ds(off, COMB_CHUNK)], r3_v, s3)
            d1 = pltpu.async_copy(y_hbm.at[i1_v], r1_v, s1)
            d2 = pltpu.async_copy(y_hbm.at[i2_v], r2_v, s2)
            d1.wait()
            d2.wait()
            d3.wait()

            def add_row(r, c):
                for j in range(nvec):
                    sl = pl.ds(j * 16, 16)
                    r1_v[r, sl] = r1_v[r, sl] + r2_v[r, sl] + r3_v[r, sl]
                return c

            lax.fori_loop(0, COMB_CHUNK, add_row, 0)
            pltpu.sync_copy(r1_v, out_hbm.at[pl.ds(off, COMB_CHUNK)])
            return carry

        lax.fori_loop(0, nch, body, 0)

    return comb(y_s, y_shared, pos1, pos2)


def kernel(x, gate_w, ew1, eb1, ew2, eb2, sw1, sb1, sw2, sb2):
    b, t, h, w, hid = x.shape
    n = b * t * h * w
    e = gate_w.shape[0]
    inter = ew1.shape[2]
    x_flat = x.reshape(n, hid)

    i1, v1, r1, i2, v2, r2, t1, t2 = _gating(x_flat, gate_w)
    i1, i2 = i1[:, 0], i2[:, 0]
    r1, r2 = r1[:, 0], r2[:, 0]
    tot1 = t1[0].astype(I32)
    counts = (t1[0] + t2[0]).astype(I32)

    blocks_e = (counts + BLK - 1) // BLK
    block_base = (BLK * (jnp.cumsum(blocks_e) - blocks_e)).astype(I32)
    pos1 = block_base[i1] + r1
    pos2 = block_base[i2] + tot1[i2] + r2
    pos_pair = jnp.concatenate([pos1, pos2])
    wts = jnp.concatenate([v1, v2])[:, 0]

    cap = 2 * n + e * BLK
    w_ext = jnp.zeros((cap,), F32).at[pos_pair].set(
        wts, unique_indices=True, mode="drop").reshape(cap, 1)
    nbr = cap // BLK
    seg_ends = jnp.cumsum(blocks_e)
    block_expert = jnp.minimum(
        jnp.searchsorted(seg_ends, jnp.arange(nbr), side="right"),
        e - 1).astype(I32)

    bf16 = jnp.bfloat16
    w1_all = ew1.astype(bf16)
    b1_all = eb1[:, None, :]
    w2_all = ew2.astype(bf16)
    b2_all = eb2[:, None, :]

    x_sorted = _dispatch(x_flat, pos_pair, cap)

    y_s = _grouped_mlp(x_sorted, block_expert, w1_all, b1_all, w2_all,
                       b2_all, w_ext)
    y_sh = _shared_mlp(x_flat, sw1.astype(bf16), sb1, sw2.astype(bf16),
                       sb2)

    out = _combine(y_s, y_sh, pos1, pos2)
    return out.reshape(b, t, h, w, hid)


# speedup vs baseline: 3.2091x; 1.0985x over previous
import functools

import jax
import jax.numpy as jnp
from jax import lax
from jax.experimental import pallas as pl
from jax.experimental.pallas import tpu as pltpu
from jax.experimental.pallas import tpu_sc as plsc

F32 = jnp.float32
I32 = jnp.int32

BLK = 256
GATE_BLK = 1024
NW = 32
DISP_CHUNK = 64
COMB_CHUNK = 32


def _gate_body(x_ref, gw_ref, i1_ref, v1_ref, r1_ref, i2_ref, v2_ref, r2_ref,
               t1_ref, t2_ref, c1_ref, c2_ref):
    b = pl.program_id(0)

    @pl.when(b == 0)
    def _init():
        c1_ref[...] = jnp.zeros_like(c1_ref)
        c2_ref[...] = jnp.zeros_like(c2_ref)

    x = x_ref[...]
    logits = lax.dot_general(x, gw_ref[...], (((1,), (1,)), ((), ())),
                             preferred_element_type=F32)
    gb, e = logits.shape
    m = jnp.max(logits, axis=-1, keepdims=True)
    ex = jnp.exp(logits - m)
    scores = ex / jnp.sum(ex, axis=-1, keepdims=True)
    ii = lax.broadcasted_iota(I32, (gb, e), 1)
    v1 = jnp.max(scores, axis=-1, keepdims=True)
    i1 = jnp.min(jnp.where(scores == v1, ii, e), axis=-1, keepdims=True)
    rest = jnp.where(ii == i1, -1.0, scores)
    v2 = jnp.max(rest, axis=-1, keepdims=True)
    i2 = jnp.min(jnp.where(rest == v2, ii, e), axis=-1, keepdims=True)
    wsum = v1 + v2

    onehot1 = (ii == i1).astype(F32)
    onehot2 = (ii == i2).astype(F32)
    lt = (lax.broadcasted_iota(I32, (gb, gb), 0)
          > lax.broadcasted_iota(I32, (gb, gb), 1)).astype(F32)
    prefix1 = jnp.dot(lt, onehot1, preferred_element_type=F32)
    prefix2 = jnp.dot(lt, onehot2, preferred_element_type=F32)
    c1 = c1_ref[...]
    c2 = c2_ref[...]
    rank1 = jnp.sum(jnp.where(onehot1 > 0, prefix1 + c1, 0.0),
                    axis=-1, keepdims=True)
    rank2 = jnp.sum(jnp.where(onehot2 > 0, prefix2 + c2, 0.0),
                    axis=-1, keepdims=True)
    c1_ref[...] = c1 + jnp.sum(onehot1, axis=0, keepdims=True)
    c2_ref[...] = c2 + jnp.sum(onehot2, axis=0, keepdims=True)

    i1_ref[...] = i1
    i2_ref[...] = i2
    v1_ref[...] = v1 / wsum
    v2_ref[...] = v2 / wsum
    r1_ref[...] = rank1.astype(I32)
    r2_ref[...] = rank2.astype(I32)
    t1_ref[...] = c1_ref[...]
    t2_ref[...] = c2_ref[...]


def _gating(x_flat, gate_w):
    n, hid = x_flat.shape
    e = gate_w.shape[0]
    col = lambda b: (b, 0)
    whole = lambda b: (0, 0)
    return pl.pallas_call(
        _gate_body,
        grid=(n // GATE_BLK,),
        in_specs=[
            pl.BlockSpec((GATE_BLK, hid), col),
            pl.BlockSpec((e, hid), whole),
        ],
        out_specs=[
            pl.BlockSpec((GATE_BLK, 1), col),
            pl.BlockSpec((GATE_BLK, 1), col),
            pl.BlockSpec((GATE_BLK, 1), col),
            pl.BlockSpec((GATE_BLK, 1), col),
            pl.BlockSpec((GATE_BLK, 1), col),
            pl.BlockSpec((GATE_BLK, 1), col),
            pl.BlockSpec((1, e), whole),
            pl.BlockSpec((1, e), whole),
        ],
        out_shape=[
            jax.ShapeDtypeStruct((n, 1), I32),
            jax.ShapeDtypeStruct((n, 1), F32),
            jax.ShapeDtypeStruct((n, 1), I32),
            jax.ShapeDtypeStruct((n, 1), I32),
            jax.ShapeDtypeStruct((n, 1), F32),
            jax.ShapeDtypeStruct((n, 1), I32),
            jax.ShapeDtypeStruct((1, e), F32),
            jax.ShapeDtypeStruct((1, e), F32),
        ],
        scratch_shapes=[
            pltpu.VMEM((1, e), F32),
            pltpu.VMEM((1, e), F32),
        ],
    )(x_flat, gate_w)


def _moe_body(be_ref, x_ref, w1_ref, b1_ref, w2_ref, b2_ref, ws_ref, y_ref):
    del be_ref
    x = x_ref[...]
    h = jnp.dot(x, w1_ref[0], preferred_element_type=F32) + b1_ref[0]
    h = 0.5 * h * (1.0 + lax.erf(h * 0.7071067811865476))
    y = jnp.dot(h, w2_ref[0], preferred_element_type=F32) + b2_ref[0]
    y_ref[...] = y * ws_ref[...]


def _grouped_mlp(x_sorted, block_expert, w1_all, b1_all, w2_all, b2_all, w_ext):
    capx, hid = x_sorted.shape
    ne, _, inter = w1_all.shape
    nb = capx // BLK
    grid_spec = pltpu.PrefetchScalarGridSpec(
        num_scalar_prefetch=1,
        grid=(nb,),
        in_specs=[
            pl.BlockSpec((BLK, hid), lambda b, be: (b, 0)),
            pl.BlockSpec((1, hid, inter), lambda b, be: (be[b], 0, 0)),
            pl.BlockSpec((1, 1, inter), lambda b, be: (be[b], 0, 0)),
            pl.BlockSpec((1, inter, hid), lambda b, be: (be[b], 0, 0)),
            pl.BlockSpec((1, 1, hid), lambda b, be: (be[b], 0, 0)),
            pl.BlockSpec((BLK, 1), lambda b, be: (b, 0)),
        ],
        out_specs=pl.BlockSpec((BLK, hid), lambda b, be: (b, 0)),
    )
    return pl.pallas_call(
        _moe_body,
        grid_spec=grid_spec,
        out_shape=jax.ShapeDtypeStruct((capx, hid), F32),
    )(block_expert, x_sorted, w1_all, b1_all, w2_all, b2_all, w_ext)


def _shared_body(x_ref, w1_ref, b1_ref, w2_ref, b2_ref, y_ref):
    x = x_ref[...]
    h = jnp.dot(x, w1_ref[...], preferred_element_type=F32) + b1_ref[...]
    h = 0.5 * h * (1.0 + lax.erf(h * 0.7071067811865476))
    y_ref[...] = jnp.dot(h, w2_ref[...],
                         preferred_element_type=F32) + b2_ref[...]


def _shared_mlp(x_flat, sw1, sb1, sw2, sb2):
    n, hid = x_flat.shape
    inter = sw1.shape[1]
    return pl.pallas_call(
        _shared_body,
        grid=(n // BLK,),
        in_specs=[
            pl.BlockSpec((BLK, hid), lambda b: (b, 0)),
            pl.BlockSpec((hid, inter), lambda b: (0, 0)),
            pl.BlockSpec((1, inter), lambda b: (0, 0)),
            pl.BlockSpec((inter, hid), lambda b: (0, 0)),
            pl.BlockSpec((1, hid), lambda b: (0, 0)),
        ],
        out_specs=pl.BlockSpec((BLK, hid), lambda b: (b, 0)),
        out_shape=jax.ShapeDtypeStruct((n, hid), F32),
    )(x_flat, sw1, sb1[None, :], sw2, sb2[None, :])


def _dispatch(x_flat, pos_pair, cap):
    n, hid = x_flat.shape
    m2 = pos_pair.shape[0]
    per_w = m2 // NW
    nch = per_w // DISP_CHUNK
    mesh = plsc.VectorSubcoreMesh(core_axis_name="c", subcore_axis_name="s")

    @functools.partial(
        pl.kernel,
        mesh=mesh,
        out_type=jax.ShapeDtypeStruct((cap, hid), F32),
        scratch_types=[
            pltpu.VMEM((DISP_CHUNK,), I32),
            pltpu.VMEM((DISP_CHUNK, hid), F32),
            pltpu.SemaphoreType.DMA,
        ],
    )
    def disp(x_hbm, pos_hbm, out_hbm, idx_v, rows_v, semx):
        wid = lax.axis_index("s") * 2 + lax.axis_index("c")
        base = wid * per_w

        def body(i, carry):
            off = pl.multiple_of(base + i * DISP_CHUNK, DISP_CHUNK)
            xoff = pl.multiple_of(lax.rem(off, n), DISP_CHUNK)
            pltpu.sync_copy(pos_hbm.at[pl.ds(off, DISP_CHUNK)], idx_v)
            pltpu.sync_copy(x_hbm.at[pl.ds(xoff, DISP_CHUNK)], rows_v)
            pltpu.async_copy(rows_v, out_hbm.at[idx_v], semx).wait()
            return carry

        lax.fori_loop(0, nch, body, 0)

    return disp(x_flat, pos_pair)


def _combine(y_s, y_shared, pos1, pos2):
    capx, hid = y_s.shape
    n = pos1.shape[0]
    per_w = n // NW
    nch = per_w // COMB_CHUNK
    mesh = plsc.VectorSubcoreMesh(core_axis_name="c", subcore_axis_name="s")
    nvec = hid // 16

    @functools.partial(
        pl.kernel,
        mesh=mesh,
        out_type=jax.ShapeDtypeStruct((n, hid), F32),
        scratch_types=[
            pltpu.VMEM((COMB_CHUNK,), I32),
            pltpu.VMEM((COMB_CHUNK,), I32),
            pltpu.VMEM((COMB_CHUNK, hid), F32),
            pltpu.VMEM((COMB_CHUNK, hid), F32),
            pltpu.VMEM((COMB_CHUNK, hid), F32),
            pltpu.SemaphoreType.DMA,
            pltpu.SemaphoreType.DMA,
            pltpu.SemaphoreType.DMA,
        ],
    )
    def comb(y_hbm, ysh_hbm, p1_hbm, p2_hbm, out_hbm,
             i1_v, i2_v, r1_v, r2_v, r3_v, s1, s2, s3):
        wid = lax.axis_index("s") * 2 + lax.axis_index("c")
        base = wid * per_w

        def body(i, carry):
            off = pl.multiple_of(base + i * COMB_CHUNK, COMB_CHUNK)
            pltpu.sync_copy(p1_hbm.at[pl.ds(off, COMB_CHUNK)], i1_v)
            pltpu.sync_copy(p2_hbm.at[pl.ds(off, COMB_CHUNK)], i2_v)
            d3 = pltpu.async_copy(ysh_hbm.at[pl.ds(off, COMB_CHUNK)], r3_v, s3)
            d1 = pltpu.async_copy(y_hbm.at[i1_v], r1_v, s1)
            d2 = pltpu.async_copy(y_hbm.at[i2_v], r2_v, s2)
            d1.wait()
            d2.wait()
            d3.wait()

            def add_row(r, c):
                for j in range(nvec):
                    sl = pl.ds(j * 16, 16)
                    r1_v[r, sl] = r1_v[r, sl] + r2_v[r, sl] + r3_v[r, sl]
                return c

            lax.fori_loop(0, COMB_CHUNK, add_row, 0)
            pltpu.sync_copy(r1_v, out_hbm.at[pl.ds(off, COMB_CHUNK)])
            return carry

        lax.fori_loop(0, nch, body, 0)

    return comb(y_s, y_shared, pos1, pos2)


def kernel(x, gate_w, ew1, eb1, ew2, eb2, sw1, sb1, sw2, sb2):
    b, t, h, w, hid = x.shape
    n = b * t * h * w
    e = gate_w.shape[0]
    inter = ew1.shape[2]
    x_flat = x.reshape(n, hid)

    i1, v1, r1, i2, v2, r2, t1, t2 = _gating(x_flat, gate_w)
    i1, i2 = i1[:, 0], i2[:, 0]
    r1, r2 = r1[:, 0], r2[:, 0]
    tot1 = t1[0].astype(I32)
    counts = (t1[0] + t2[0]).astype(I32)

    blocks_e = (counts + BLK - 1) // BLK
    block_base = (BLK * (jnp.cumsum(blocks_e) - blocks_e)).astype(I32)
    pos1 = block_base[i1] + r1
    pos2 = block_base[i2] + tot1[i2] + r2
    pos_pair = jnp.concatenate([pos1, pos2])
    wts = jnp.concatenate([v1, v2])[:, 0]

    cap = 2 * n + e * BLK
    w_ext = jnp.zeros((cap,), F32).at[pos_pair].set(
        wts, unique_indices=True, mode="drop").reshape(cap, 1)
    nbr = cap // BLK
    seg_ends = jnp.cumsum(blocks_e)
    block_expert = jnp.minimum(
        jnp.searchsorted(seg_ends, jnp.arange(nbr), side="right"),
        e - 1).astype(I32)

    w1_all = ew1
    b1_all = eb1[:, None, :]
    w2_all = ew2
    b2_all = eb2[:, None, :]

    x_sorted = _dispatch(x_flat, pos_pair, cap)

    y_s = _grouped_mlp(x_sorted, block_expert, w1_all, b1_all, w2_all,
                       b2_all, w_ext)
    y_sh = _shared_mlp(x_flat, sw1, sb1, sw2, sb2)

    out = _combine(y_s, y_sh, pos1, pos2)
    return out.reshape(b, t, h, w, hid)


# speedup vs baseline: 3.2977x; 1.0276x over previous
import functools

import jax
import jax.numpy as jnp
from jax import lax
from jax.experimental import pallas as pl
from jax.experimental.pallas import tpu as pltpu
from jax.experimental.pallas import tpu_sc as plsc

F32 = jnp.float32
I32 = jnp.int32

BLK = 256
GATE_BLK = 1024
NW = 32
DISP_CHUNK = 32
COMB_CHUNK = 16


def _gate_body(x_ref, gw_ref, i1_ref, v1_ref, r1_ref, i2_ref, v2_ref, r2_ref,
               t1_ref, t2_ref, c1_ref, c2_ref):
    b = pl.program_id(0)

    @pl.when(b == 0)
    def _init():
        c1_ref[...] = jnp.zeros_like(c1_ref)
        c2_ref[...] = jnp.zeros_like(c2_ref)

    x = x_ref[...]
    logits = lax.dot_general(x, gw_ref[...], (((1,), (1,)), ((), ())),
                             preferred_element_type=F32)
    gb, e = logits.shape
    m = jnp.max(logits, axis=-1, keepdims=True)
    ex = jnp.exp(logits - m)
    scores = ex / jnp.sum(ex, axis=-1, keepdims=True)
    ii = lax.broadcasted_iota(I32, (gb, e), 1)
    v1 = jnp.max(scores, axis=-1, keepdims=True)
    i1 = jnp.min(jnp.where(scores == v1, ii, e), axis=-1, keepdims=True)
    rest = jnp.where(ii == i1, -1.0, scores)
    v2 = jnp.max(rest, axis=-1, keepdims=True)
    i2 = jnp.min(jnp.where(rest == v2, ii, e), axis=-1, keepdims=True)
    wsum = v1 + v2

    onehot1 = (ii == i1).astype(F32)
    onehot2 = (ii == i2).astype(F32)
    lt = (lax.broadcasted_iota(I32, (gb, gb), 0)
          > lax.broadcasted_iota(I32, (gb, gb), 1)).astype(F32)
    prefix1 = jnp.dot(lt, onehot1, preferred_element_type=F32)
    prefix2 = jnp.dot(lt, onehot2, preferred_element_type=F32)
    c1 = c1_ref[...]
    c2 = c2_ref[...]
    rank1 = jnp.sum(jnp.where(onehot1 > 0, prefix1 + c1, 0.0),
                    axis=-1, keepdims=True)
    rank2 = jnp.sum(jnp.where(onehot2 > 0, prefix2 + c2, 0.0),
                    axis=-1, keepdims=True)
    c1_ref[...] = c1 + jnp.sum(onehot1, axis=0, keepdims=True)
    c2_ref[...] = c2 + jnp.sum(onehot2, axis=0, keepdims=True)

    i1_ref[...] = i1
    i2_ref[...] = i2
    v1_ref[...] = v1 / wsum
    v2_ref[...] = v2 / wsum
    r1_ref[...] = rank1.astype(I32)
    r2_ref[...] = rank2.astype(I32)
    t1_ref[...] = c1_ref[...]
    t2_ref[...] = c2_ref[...]


def _gating(x_flat, gate_w):
    n, hid = x_flat.shape
    e = gate_w.shape[0]
    col = lambda b: (b, 0)
    whole = lambda b: (0, 0)
    return pl.pallas_call(
        _gate_body,
        grid=(n // GATE_BLK,),
        in_specs=[
            pl.BlockSpec((GATE_BLK, hid), col),
            pl.BlockSpec((e, hid), whole),
        ],
        out_specs=[
            pl.BlockSpec((GATE_BLK, 1), col),
            pl.BlockSpec((GATE_BLK, 1), col),
            pl.BlockSpec((GATE_BLK, 1), col),
            pl.BlockSpec((GATE_BLK, 1), col),
            pl.BlockSpec((GATE_BLK, 1), col),
            pl.BlockSpec((GATE_BLK, 1), col),
            pl.BlockSpec((1, e), whole),
            pl.BlockSpec((1, e), whole),
        ],
        out_shape=[
            jax.ShapeDtypeStruct((n, 1), I32),
            jax.ShapeDtypeStruct((n, 1), F32),
            jax.ShapeDtypeStruct((n, 1), I32),
            jax.ShapeDtypeStruct((n, 1), I32),
            jax.ShapeDtypeStruct((n, 1), F32),
            jax.ShapeDtypeStruct((n, 1), I32),
            jax.ShapeDtypeStruct((1, e), F32),
            jax.ShapeDtypeStruct((1, e), F32),
        ],
        scratch_shapes=[
            pltpu.VMEM((1, e), F32),
            pltpu.VMEM((1, e), F32),
        ],
    )(x_flat, gate_w)


def _moe_body(be_ref, x_ref, w1_ref, b1_ref, w2_ref, b2_ref, ws_ref, y_ref):
    del be_ref
    x = x_ref[...]
    h = jnp.dot(x, w1_ref[0], preferred_element_type=F32) + b1_ref[0]
    h = 0.5 * h * (1.0 + lax.erf(h * 0.7071067811865476))
    y = jnp.dot(h, w2_ref[0], preferred_element_type=F32) + b2_ref[0]
    y_ref[...] = y * ws_ref[...]


def _grouped_mlp(x_sorted, block_expert, w1_all, b1_all, w2_all, b2_all, w_ext):
    capx, hid = x_sorted.shape
    ne, _, inter = w1_all.shape
    nb = capx // BLK
    grid_spec = pltpu.PrefetchScalarGridSpec(
        num_scalar_prefetch=1,
        grid=(nb,),
        in_specs=[
            pl.BlockSpec((BLK, hid), lambda b, be: (b, 0)),
            pl.BlockSpec((1, hid, inter), lambda b, be: (be[b], 0, 0)),
            pl.BlockSpec((1, 1, inter), lambda b, be: (be[b], 0, 0)),
            pl.BlockSpec((1, inter, hid), lambda b, be: (be[b], 0, 0)),
            pl.BlockSpec((1, 1, hid), lambda b, be: (be[b], 0, 0)),
            pl.BlockSpec((BLK, 1), lambda b, be: (b, 0)),
        ],
        out_specs=pl.BlockSpec((BLK, hid), lambda b, be: (b, 0)),
    )
    return pl.pallas_call(
        _moe_body,
        grid_spec=grid_spec,
        out_shape=jax.ShapeDtypeStruct((capx, hid), F32),
    )(block_expert, x_sorted, w1_all, b1_all, w2_all, b2_all, w_ext)


def _shared_body(x_ref, w1_ref, b1_ref, w2_ref, b2_ref, y_ref):
    x = x_ref[...]
    h = jnp.dot(x, w1_ref[...], preferred_element_type=F32) + b1_ref[...]
    h = 0.5 * h * (1.0 + lax.erf(h * 0.7071067811865476))
    y_ref[...] = jnp.dot(h, w2_ref[...],
                         preferred_element_type=F32) + b2_ref[...]


def _shared_mlp(x_flat, sw1, sb1, sw2, sb2):
    n, hid = x_flat.shape
    inter = sw1.shape[1]
    return pl.pallas_call(
        _shared_body,
        grid=(n // BLK,),
        in_specs=[
            pl.BlockSpec((BLK, hid), lambda b: (b, 0)),
            pl.BlockSpec((hid, inter), lambda b: (0, 0)),
            pl.BlockSpec((1, inter), lambda b: (0, 0)),
            pl.BlockSpec((inter, hid), lambda b: (0, 0)),
            pl.BlockSpec((1, hid), lambda b: (0, 0)),
        ],
        out_specs=pl.BlockSpec((BLK, hid), lambda b: (b, 0)),
        out_shape=jax.ShapeDtypeStruct((n, hid), F32),
    )(x_flat, sw1, sb1[None, :], sw2, sb2[None, :])


def _dispatch(x_flat, pos_pair, cap):
    n, hid = x_flat.shape
    m2 = pos_pair.shape[0]
    per_w = m2 // NW
    nch = per_w // DISP_CHUNK
    mesh = plsc.VectorSubcoreMesh(core_axis_name="c", subcore_axis_name="s")

    @functools.partial(
        pl.kernel,
        mesh=mesh,
        out_type=jax.ShapeDtypeStruct((cap, hid), F32),
        scratch_types=[
            pltpu.VMEM((DISP_CHUNK,), I32),
            pltpu.VMEM((DISP_CHUNK, hid), F32),
            pltpu.VMEM((DISP_CHUNK,), I32),
            pltpu.VMEM((DISP_CHUNK, hid), F32),
            pltpu.SemaphoreType.DMA,
        ],
    )
    def disp(x_hbm, pos_hbm, out_hbm, idx_a, rows_a, idx_b, rows_b, semx):
        wid = lax.axis_index("s") * 2 + lax.axis_index("c")
        base = wid * per_w

        def load(i, idx_v, rows_v):
            off = pl.multiple_of(base + i * DISP_CHUNK, DISP_CHUNK)
            xoff = pl.multiple_of(lax.rem(off, n), DISP_CHUNK)
            pltpu.sync_copy(pos_hbm.at[pl.ds(off, DISP_CHUNK)], idx_v)
            pltpu.sync_copy(x_hbm.at[pl.ds(xoff, DISP_CHUNK)], rows_v)

        def step(i, idx_v, rows_v, idx_n, rows_n):
            d = pltpu.async_copy(rows_v, out_hbm.at[idx_v], semx)

            @pl.when(i + 1 < nch)
            def _():
                load(i + 1, idx_n, rows_n)

            d.wait()

        load(0, idx_a, rows_a)

        def body(p, carry):
            step(2 * p, idx_a, rows_a, idx_b, rows_b)
            step(2 * p + 1, idx_b, rows_b, idx_a, rows_a)
            return carry

        lax.fori_loop(0, nch // 2, body, 0)

    return disp(x_flat, pos_pair)


def _combine(y_s, y_shared, pos1, pos2):
    capx, hid = y_s.shape
    n = pos1.shape[0]
    per_w = n // NW
    nch = per_w // COMB_CHUNK
    mesh = plsc.VectorSubcoreMesh(core_axis_name="c", subcore_axis_name="s")
    nvec = hid // 16

    buf = lambda: pltpu.VMEM((COMB_CHUNK, hid), F32)
    idxb = lambda: pltpu.VMEM((COMB_CHUNK,), I32)

    @functools.partial(
        pl.kernel,
        mesh=mesh,
        out_type=jax.ShapeDtypeStruct((n, hid), F32),
        scratch_types=[
            idxb(), idxb(), buf(), buf(), buf(),
            idxb(), idxb(), buf(), buf(), buf(),
            pltpu.SemaphoreType.DMA,
            pltpu.SemaphoreType.DMA,
            pltpu.SemaphoreType.DMA,
            pltpu.SemaphoreType.DMA,
            pltpu.SemaphoreType.DMA,
            pltpu.SemaphoreType.DMA,
        ],
    )
    def comb(y_hbm, ysh_hbm, p1_hbm, p2_hbm, out_hbm,
             i1_a, i2_a, r1_a, r2_a, r3_a,
             i1_b, i2_b, r1_b, r2_b, r3_b,
             s1a, s2a, s3a, s1b, s2b, s3b):
        wid = lax.axis_index("s") * 2 + lax.axis_index("c")
        base = wid * per_w

        def fire(i, i1_v, i2_v, r1_v, r2_v, r3_v, s1, s2, s3):
            off = pl.multiple_of(base + i * COMB_CHUNK, COMB_CHUNK)
            pltpu.sync_copy(p1_hbm.at[pl.ds(off, COMB_CHUNK)], i1_v)
            pltpu.sync_copy(p2_hbm.at[pl.ds(off, COMB_CHUNK)], i2_v)
            pltpu.async_copy(ysh_hbm.at[pl.ds(off, COMB_CHUNK)], r3_v, s3)
            pltpu.async_copy(y_hbm.at[i1_v], r1_v, s1)
            pltpu.async_copy(y_hbm.at[i2_v], r2_v, s2)

        def drain_add_store(i, i1_v, i2_v, r1_v, r2_v, r3_v, s1, s2, s3):
            off = pl.multiple_of(base + i * COMB_CHUNK, COMB_CHUNK)
            pltpu.make_async_copy(y_hbm.at[i1_v], r1_v, s1).wait()
            pltpu.make_async_copy(y_hbm.at[i2_v], r2_v, s2).wait()
            pltpu.make_async_copy(
                ysh_hbm.at[pl.ds(off, COMB_CHUNK)], r3_v, s3).wait()

            def add_row(r, c):
                for j in range(nvec):
                    sl = pl.ds(j * 16, 16)
                    r1_v[r, sl] = r1_v[r, sl] + r2_v[r, sl] + r3_v[r, sl]
                return c

            lax.fori_loop(0, COMB_CHUNK, add_row, 0)
            pltpu.sync_copy(r1_v, out_hbm.at[pl.ds(off, COMB_CHUNK)])

        seta = (i1_a, i2_a, r1_a, r2_a, r3_a, s1a, s2a, s3a)
        setb = (i1_b, i2_b, r1_b, r2_b, r3_b, s1b, s2b, s3b)

        fire(0, *seta)

        def body(p, carry):
            i = 2 * p

            @pl.when(i + 1 < nch)
            def _():
                fire(i + 1, *setb)

            drain_add_store(i, *seta)

            @pl.when(i + 2 < nch)
            def _():
                fire(i + 2, *seta)

            @pl.when(i + 1 < nch)
            def _():
                drain_add_store(i + 1, *setb)

            return carry

        lax.fori_loop(0, (nch + 1) // 2, body, 0)

    return comb(y_s, y_shared, pos1, pos2)


def kernel(x, gate_w, ew1, eb1, ew2, eb2, sw1, sb1, sw2, sb2):
    b, t, h, w, hid = x.shape
    n = b * t * h * w
    e = gate_w.shape[0]
    inter = ew1.shape[2]
    x_flat = x.reshape(n, hid)

    i1, v1, r1, i2, v2, r2, t1, t2 = _gating(x_flat, gate_w)
    i1, i2 = i1[:, 0], i2[:, 0]
    r1, r2 = r1[:, 0], r2[:, 0]
    tot1 = t1[0].astype(I32)
    counts = (t1[0] + t2[0]).astype(I32)

    blocks_e = (counts + BLK - 1) // BLK
    block_base = (BLK * (jnp.cumsum(blocks_e) - blocks_e)).astype(I32)
    pos1 = block_base[i1] + r1
    pos2 = block_base[i2] + tot1[i2] + r2
    pos_pair = jnp.concatenate([pos1, pos2])
    wts = jnp.concatenate([v1, v2])[:, 0]

    cap = 2 * n + e * BLK
    w_ext = jnp.zeros((cap,), F32).at[pos_pair].set(
        wts, unique_indices=True, mode="drop").reshape(cap, 1)
    nbr = cap // BLK
    seg_ends = jnp.cumsum(blocks_e)
    block_expert = jnp.minimum(
        jnp.searchsorted(seg_ends, jnp.arange(nbr), side="right"),
        e - 1).astype(I32)

    w1_all = ew1
    b1_all = eb1[:, None, :]
    w2_all = ew2
    b2_all = eb2[:, None, :]

    x_sorted = _dispatch(x_flat, pos_pair, cap)

    y_s = _grouped_mlp(x_sorted, block_expert, w1_all, b1_all, w2_all,
                       b2_all, w_ext)
    y_sh = _shared_mlp(x_flat, sw1, sb1, sw2, sb2)

    out = _combine(y_s, y_sh, pos1, pos2)
    return out.reshape(b, t, h, w, hid)
